# Initial kernel scaffold; baseline (speedup 1.0000x reference)
import jax
import jax.numpy as jnp
from jax.experimental import pallas as pl


def kernel(x, edge_index, batch, reconstruct, t1_W, t1_b, t2_W, t2_b, W1s, b1s, gammas, betas, W2s, b2s):
    raise NotImplementedError("write your pallas kernel here")


# speedup vs baseline: 2.9207x; 2.9207x over previous
import functools

import jax
import jax.numpy as jnp
from jax import lax
from jax.experimental import pallas as pl
from jax.experimental.pallas import tpu as pltpu
from jax.experimental.pallas import tpu_sc as plsc

N = 10000
E = 320000
F = 128
H = 128
G = 128

CHUNK = 128
NCH = 160
NCH0 = 40
EPAD = 16 * NCH * CHUNK
NPAD = 10112
STRIPE = NPAD // 16


def _mm_t(a, b):
    return lax.dot_general(a, b, (((1,), (1,)), ((), ())),
                           precision=lax.Precision.HIGHEST,
                           preferred_element_type=jnp.float32)


def _in_mlp_body(x_ref, w_ref, b_ref, o_ref):
    o_ref[...] = jnp.maximum(_mm_t(x_ref[...], w_ref[...]) + b_ref[...], 0.0)


def _layer_body(h_ref, a_ref, w1_ref, b1_ref, g_ref, be_ref, w2_ref, b2_ref,
                o_ref):
    u = h_ref[...] + a_ref[0, :N] + a_ref[1, :N]
    z = _mm_t(u, w1_ref[...]) + b1_ref[...]
    mean = jnp.mean(z, axis=0, keepdims=True)
    var = jnp.mean(z * z, axis=0, keepdims=True) - mean * mean
    zn = (z - mean) * (lax.rsqrt(var + 1e-5) * g_ref[...]) + be_ref[...]
    zn = jnp.maximum(zn, 0.0)
    o_ref[...] = _mm_t(zn, w2_ref[...]) + b2_ref[...]


def _final_body(h_ref, w_ref, b_ref, batch_ref, o_ref):
    y = _mm_t(h_ref[...], w_ref[...]) + b_ref[...]
    seg = batch_ref[0, :]
    onehot = (seg[None, :] == lax.broadcasted_iota(jnp.int32, (G, N), 0))
    onehot = onehot.astype(jnp.float32)
    o_ref[...] = lax.dot_general(onehot, y, (((1,), (0,)), ((), ())),
                                 precision=lax.Precision.HIGHEST,
                                 preferred_element_type=jnp.float32)


def _full(shape):
    return pl.BlockSpec(shape, lambda: tuple(0 for _ in shape))


def _in_mlp(x, w, b):
    return pl.pallas_call(
        _in_mlp_body,
        in_specs=[_full((N, F)), _full((H, F)), _full((1, H))],
        out_specs=_full((N, H)),
        out_shape=jax.ShapeDtypeStruct((N, H), jnp.float32),
    )(x, w, b)


def _layer_tc(h, aggs, w1, b1, gamma, beta, w2, b2):
    return pl.pallas_call(
        _layer_body,
        in_specs=[_full((N, H)), _full((2, NPAD, H)), _full((H, H)),
                  _full((1, H)), _full((1, H)), _full((1, H)), _full((H, H)),
                  _full((1, H))],
        out_specs=_full((N, H)),
        out_shape=jax.ShapeDtypeStruct((N, H), jnp.float32),
    )(h, aggs, w1, b1, gamma, beta, w2, b2)


def _final(h, w, b, batch2):
    return pl.pallas_call(
        _final_body,
        in_specs=[_full((N, H)), _full((H, H)), _full((1, H)),
                  _full((1, N))],
        out_specs=_full((G, H)),
        out_shape=jax.ShapeDtypeStruct((G, H), jnp.float32),
    )(h, w, b, batch2)


def _sc_agg(h, src3, dst3, zeros):
    mesh = plsc.VectorSubcoreMesh(core_axis_name="c", subcore_axis_name="s")

    @functools.partial(
        pl.kernel,
        mesh=mesh,
        out_type=jax.ShapeDtypeStruct((2, NPAD, H), jnp.float32),
        scratch_types=[
            pltpu.VMEM((64, CHUNK), jnp.int32),
            pltpu.VMEM((64, CHUNK), jnp.int32),
            pltpu.VMEM((CHUNK, H), jnp.float32),
            pltpu.VMEM((CHUNK, H), jnp.float32),
            pltpu.VMEM_SHARED((NPAD, H), jnp.float32),
            pltpu.SemaphoreType.DMA,
            pltpu.SemaphoreType.DMA,
        ],
    )
    def body(h_hbm, src_hbm, dst_hbm, zeros_hbm, out_hbm,
             sidx, didx, rows0, rows1, agg, sem0, sem1):
        c = lax.axis_index("c")
        s = lax.axis_index("s")
        pltpu.sync_copy(zeros_hbm, agg.at[pl.ds(s * STRIPE, STRIPE)])
        plsc.subcore_barrier()

        def stage_and_run(off, cnt):
            pltpu.sync_copy(src_hbm.at[s, pl.ds(off, cnt)],
                            sidx.at[pl.ds(0, cnt)])
            pltpu.sync_copy(dst_hbm.at[s, pl.ds(off, cnt)],
                            didx.at[pl.ds(0, cnt)])
            npair = cnt // 2
            pltpu.async_copy(h_hbm.at[sidx.at[0]], rows0, sem0)

            def pair(jj, c2):
                j0 = 2 * jj
                j1 = j0 + 1
                pltpu.async_copy(h_hbm.at[sidx.at[j1]], rows1, sem1)
                pltpu.make_async_copy(h_hbm.at[sidx.at[j0]], rows0,
                                      sem0).wait()
                pltpu.sync_copy(rows0, agg.at[didx.at[j0]], add=True)

                @pl.when(jj + 1 < npair)
                def _():
                    pltpu.async_copy(h_hbm.at[sidx.at[j0 + 2]], rows0, sem0)

                pltpu.make_async_copy(h_hbm.at[sidx.at[j1]], rows1,
                                      sem1).wait()
                pltpu.sync_copy(rows1, agg.at[didx.at[j1]], add=True)
                return c2

            lax.fori_loop(0, npair, pair, 0)

        @pl.when(c == 0)
        def _():
            stage_and_run(0, NCH0)

        @pl.when(c == 1)
        def _():
            stage_and_run(NCH0, 56)
            stage_and_run(96, 64)

        plsc.subcore_barrier()
        pltpu.sync_copy(agg.at[pl.ds(s * STRIPE, STRIPE)],
                        out_hbm.at[c, pl.ds(s * STRIPE, STRIPE)])

    return body(h, src3, dst3, zeros)


def kernel(x, edge_index, batch, reconstruct, t1_W, t1_b, t2_W, t2_b,
           W1s, b1s, gammas, betas, W2s, b2s):
    del reconstruct
    pad = EPAD - E
    src = jnp.concatenate([edge_index[0], jnp.zeros((pad,), jnp.int32)])
    dst = jnp.concatenate([edge_index[1], jnp.full((pad,), N, jnp.int32)])
    src3 = src.reshape(16, NCH, CHUNK)
    dst3 = dst.reshape(16, NCH, CHUNK)
    zeros = jnp.zeros((STRIPE, H), jnp.float32)
    batch2 = batch.reshape(1, N)

    h = _in_mlp(x, t1_W, t1_b.reshape(1, H))
    for i in range(W1s.shape[0]):
        aggs = _sc_agg(h, src3, dst3, zeros)
        h = _layer_tc(h, aggs, W1s[i], b1s[i].reshape(1, H),
                      gammas[i].reshape(1, H), betas[i].reshape(1, H),
                      W2s[i], b2s[i].reshape(1, H))
    return _final(h, t2_W, t2_b.reshape(1, H), batch2)
